# baseline (speedup 1.0000x reference)
import jax
import jax.numpy as jnp
from jax.experimental import pallas as pl
from jax.experimental.pallas import tpu as pltpu

BM = 512
BN = 512
WCHUNK = 64


def _linear_kernel(x_ref, w_hbm, b_ref, o_ref, w_bf16, w_stage, w_sem):
    i = pl.program_id(0)
    n = pl.program_id(1)
    n_rows = w_bf16.shape[0]

    @pl.when((i == 0) & (n == 0))
    def _load_w():
        n_chunks = n_rows // WCHUNK

        def _start(c):
            pltpu.make_async_copy(
                w_hbm.at[pl.ds(c * WCHUNK, WCHUNK), :],
                w_stage.at[c % 2],
                w_sem.at[c % 2]).start()

        _start(0)
        _start(1)
        for c in range(n_chunks):
            h = c % 2
            pltpu.make_async_copy(
                w_hbm.at[pl.ds(c * WCHUNK, WCHUNK), :],
                w_stage.at[h],
                w_sem.at[h]).wait()
            w_bf16[pl.ds(c * WCHUNK, WCHUNK), :] = (
                w_stage[h].astype(jnp.bfloat16))
            if c + 2 < n_chunks:
                _start(c + 2)

    acc = jax.lax.dot_general(
        x_ref[...].astype(jnp.bfloat16),
        w_bf16[pl.ds(n * BN, BN), :],
        dimension_numbers=(((1,), (1,)), ((), ())),
        preferred_element_type=jnp.float32,
    )
    o_ref[...] = acc + b_ref[...]


def kernel(input, weight, bias):
    M, K = input.shape
    N = weight.shape[0]
    grid = (M // BM, N // BN)
    return pl.pallas_call(
        _linear_kernel,
        grid=grid,
        in_specs=[
            pl.BlockSpec((BM, K), lambda i, n: (i, 0)),
            pl.BlockSpec(memory_space=pl.ANY),
            pl.BlockSpec((1, BN), lambda i, n: (0, n)),
        ],
        out_specs=pl.BlockSpec((BM, BN), lambda i, n: (i, n)),
        out_shape=jax.ShapeDtypeStruct((M, N), jnp.float32),
        scratch_shapes=[
            pltpu.VMEM((N, K), jnp.bfloat16),
            pltpu.VMEM((2, WCHUNK, K), jnp.float32),
            pltpu.SemaphoreType.DMA((2,)),
        ],
        compiler_params=pltpu.CompilerParams(
            dimension_semantics=("arbitrary", "arbitrary"),
            vmem_limit_bytes=60000 * 1024,
        ),
        name="safe_linear",
    )(input, weight, bias.reshape(1, N))


# speedup vs baseline: 1.2059x; 1.2059x over previous
import jax
import jax.numpy as jnp
from jax.experimental import pallas as pl
from jax.experimental.pallas import tpu as pltpu

BM = 512
BNO = 1024
WCHUNK = 64


def _linear_kernel(x_ref, w_hbm, b_ref, o_ref, w_a, w_b, w_stage, w_sem):
    j = pl.program_id(0)
    i = pl.program_id(1)
    n = pl.program_id(2)

    half_rows = w_a.shape[0]
    n_chunks = half_rows // WCHUNK

    def _start(c, row_base):
        pltpu.make_async_copy(
            w_hbm.at[pl.ds(row_base + c * WCHUNK, WCHUNK), :],
            w_stage.at[c % 2],
            w_sem.at[c % 2]).start()

    @pl.when((j == 0) & (i == 0) & (n == 0))
    def _prologue():
        _start(0, 0)
        _start(1, 0)
        for c in range(n_chunks):
            h = c % 2
            pltpu.make_async_copy(
                w_hbm.at[pl.ds(c * WCHUNK, WCHUNK), :],
                w_stage.at[h], w_sem.at[h]).wait()
            w_a[c * WCHUNK:(c + 1) * WCHUNK, :] = w_stage[h].astype(jnp.bfloat16)
            if c + 2 < n_chunks:
                _start(c + 2, 0)
        _start(0, half_rows)
        _start(1, half_rows)

    @pl.when(j == 0)
    def _prefetch_second_half():
        s = i * 2 + n
        pltpu.make_async_copy(
            w_hbm.at[pl.ds(half_rows + s * WCHUNK, WCHUNK), :],
            w_stage.at[s % 2], w_sem.at[s % 2]).wait()
        w_b[pl.ds(s * WCHUNK, WCHUNK), :] = w_stage[s % 2].astype(jnp.bfloat16)

        @pl.when(s + 2 < n_chunks)
        def _issue_next():
            pltpu.make_async_copy(
                w_hbm.at[pl.ds(half_rows + (s + 2) * WCHUNK, WCHUNK), :],
                w_stage.at[s % 2], w_sem.at[s % 2]).start()

    for jj in (0, 1):
        for nn in (0, 1):
            @pl.when((j == jj) & (n == nn))
            def _dot(jj=jj, nn=nn):
                wref = w_a if jj == 0 else w_b
                acc = jax.lax.dot_general(
                    x_ref[...],
                    wref[nn * BNO:(nn + 1) * BNO, :],
                    dimension_numbers=(((1,), (1,)), ((), ())),
                    preferred_element_type=jnp.float32,
                )
                o_ref[...] = acc + b_ref[...]


def kernel(input, weight, bias):
    M, K = input.shape
    N = weight.shape[0]
    grid = (2, M // BM, 2)
    return pl.pallas_call(
        _linear_kernel,
        grid=grid,
        in_specs=[
            pl.BlockSpec((BM, K), lambda j, i, n: (i, 0)),
            pl.BlockSpec(memory_space=pl.ANY),
            pl.BlockSpec((1, BNO), lambda j, i, n: (0, 2 * j + n)),
        ],
        out_specs=pl.BlockSpec((BM, BNO), lambda j, i, n: (i, 2 * j + n)),
        out_shape=jax.ShapeDtypeStruct((M, N), jnp.float32),
        scratch_shapes=[
            pltpu.VMEM((N // 2, K), jnp.bfloat16),
            pltpu.VMEM((N // 2, K), jnp.bfloat16),
            pltpu.VMEM((2, WCHUNK, K), jnp.float32),
            pltpu.SemaphoreType.DMA((2,)),
        ],
        compiler_params=pltpu.CompilerParams(
            dimension_semantics=("arbitrary", "arbitrary", "arbitrary"),
            vmem_limit_bytes=60000 * 1024,
        ),
        name="safe_linear",
    )(input, weight, bias.reshape(1, N))
